# baseline (speedup 1.0000x reference)
import functools

import jax
import jax.numpy as jnp
from jax import lax
from jax.experimental import pallas as pl
from jax.experimental.pallas import tpu as pltpu
from jax.experimental.pallas import tpu_sc as plsc

_CHUNK = 128


def _gather_body(t_hbm, betas_hbm, out_hbm, idx_v, out_v, sem, *, b_per_w):
    wid = lax.axis_index("s") * 2 + lax.axis_index("c")
    base = wid * b_per_w
    n_chunks = b_per_w // _CHUNK
    pltpu.sync_copy(t_hbm.at[pl.ds(base, b_per_w)], idx_v)
    copies = [
        pltpu.async_copy(
            betas_hbm.at[idx_v.at[pl.ds(j * _CHUNK, _CHUNK)]],
            out_v.at[pl.ds(j * _CHUNK, _CHUNK)],
            sem,
        )
        for j in range(n_chunks)
    ]
    for cp in copies:
        cp.wait()
    pltpu.sync_copy(out_v, out_hbm.at[pl.ds(base, b_per_w)])


def kernel(t, betas):
    b = t.shape[0]
    n_workers = 32
    b_per_w = b // n_workers
    t32 = t.astype(jnp.int32)

    mesh = plsc.VectorSubcoreMesh(core_axis_name="c", subcore_axis_name="s")
    run = pl.kernel(
        functools.partial(_gather_body, b_per_w=b_per_w),
        mesh=mesh,
        compiler_params=pltpu.CompilerParams(needs_layout_passes=False),
        out_type=jax.ShapeDtypeStruct((b,), jnp.float32),
        scratch_types=[
            pltpu.VMEM((b_per_w,), jnp.int32),
            pltpu.VMEM((b_per_w,), jnp.float32),
            pltpu.SemaphoreType.DMA,
        ],
    )
    return run(t32, betas)


# speedup vs baseline: 1.5245x; 1.5245x over previous
import functools

import jax
import jax.numpy as jnp
from jax import lax
from jax.experimental import pallas as pl
from jax.experimental.pallas import tpu as pltpu
from jax.experimental.pallas import tpu_sc as plsc

_LANES = 16
_NUM_CORES = 1


def _gather_body(t_hbm, betas_hbm, out_hbm, tab_v, idx_v, out_v, sem_t,
                 sem_i, *, b_per_w):
    wid = lax.axis_index("s") * _NUM_CORES + lax.axis_index("c")
    base = wid * b_per_w
    cp_t = pltpu.async_copy(betas_hbm, tab_v, sem_t)
    cp_i = pltpu.async_copy(t_hbm.at[pl.ds(base, b_per_w)], idx_v, sem_i)
    cp_t.wait()
    cp_i.wait()
    for i in range(b_per_w // _LANES):
        idx = idx_v[pl.ds(i * _LANES, _LANES)]
        out_v[pl.ds(i * _LANES, _LANES)] = plsc.load_gather(tab_v, [idx])
    pltpu.sync_copy(out_v, out_hbm.at[pl.ds(base, b_per_w)])


def kernel(t, betas):
    b = t.shape[0]
    n_workers = 16 * _NUM_CORES
    b_per_w = b // n_workers
    t32 = t.astype(jnp.int32)
    v = betas.shape[0]

    mesh = plsc.VectorSubcoreMesh(
        core_axis_name="c", subcore_axis_name="s", num_cores=_NUM_CORES)
    run = pl.kernel(
        functools.partial(_gather_body, b_per_w=b_per_w),
        mesh=mesh,
        compiler_params=pltpu.CompilerParams(needs_layout_passes=False),
        out_type=jax.ShapeDtypeStruct((b,), jnp.float32),
        scratch_types=[
            pltpu.VMEM((v,), jnp.float32),
            pltpu.VMEM((b_per_w,), jnp.int32),
            pltpu.VMEM((b_per_w,), jnp.float32),
            pltpu.SemaphoreType.DMA,
            pltpu.SemaphoreType.DMA,
        ],
    )
    return run(t32, betas)


# speedup vs baseline: 1.5290x; 1.0030x over previous
import functools

import jax
import jax.numpy as jnp
from jax import lax
from jax.experimental import pallas as pl
from jax.experimental.pallas import tpu as pltpu
from jax.experimental.pallas import tpu_sc as plsc

_LANES = 16
_NUM_CORES = 1


def _gather_body(t_hbm, betas_hbm, out_hbm, tab_v, idx_v, out_v, sem_t,
                 sem_i, *, b_per_w):
    wid = lax.axis_index("s") * _NUM_CORES + lax.axis_index("c")
    base = wid * b_per_w
    cp_t = pltpu.async_copy(betas_hbm, tab_v, sem_t)
    cp_i = pltpu.async_copy(t_hbm.at[pl.ds(base, b_per_w)], idx_v, sem_i)
    cp_t.wait()
    cp_i.wait()
    def step(i, _):
        off = i * _LANES
        idx = idx_v[pl.ds(off, _LANES)]
        out_v[pl.ds(off, _LANES)] = plsc.load_gather(tab_v, [idx])
        return 0

    lax.fori_loop(0, b_per_w // _LANES, step, 0, unroll=4)
    pltpu.sync_copy(out_v, out_hbm.at[pl.ds(base, b_per_w)])


def kernel(t, betas):
    b = t.shape[0]
    n_workers = 16 * _NUM_CORES
    b_per_w = b // n_workers
    t32 = t.astype(jnp.int32)
    v = betas.shape[0]

    mesh = plsc.VectorSubcoreMesh(
        core_axis_name="c", subcore_axis_name="s", num_cores=_NUM_CORES)
    run = pl.kernel(
        functools.partial(_gather_body, b_per_w=b_per_w),
        mesh=mesh,
        compiler_params=pltpu.CompilerParams(needs_layout_passes=False),
        out_type=jax.ShapeDtypeStruct((b,), jnp.float32),
        scratch_types=[
            pltpu.VMEM((v,), jnp.float32),
            pltpu.VMEM((b_per_w,), jnp.int32),
            pltpu.VMEM((b_per_w,), jnp.float32),
            pltpu.SemaphoreType.DMA,
            pltpu.SemaphoreType.DMA,
        ],
    )
    return run(t32, betas)
